# baseline (speedup 1.0000x reference)
import functools

import jax
import jax.numpy as jnp
from jax import lax
from jax.experimental import pallas as pl
from jax.experimental.pallas import tpu as pltpu
from jax.experimental.pallas import tpu_sc as plsc

N = 10000
E = 160000
EPS = 1e-3
F = 48
NPAD = 10240
NC, NS = 2, 16
NW = NC * NS
EPT = 5120
EPAD = EPT * NW
SUB = 128
NSB = EPT // SUB
CH = 512
CHSB = CH // SUB
CHB = CH
ROWS_PER = NPAD // NS

_MESH = plsc.VectorSubcoreMesh(core_axis_name="c", subcore_axis_name="s")


def _sanitize_dst(idxd_v, wid):
    iota16 = lax.iota(jnp.int32, 16)

    def body(g, carry):
        r = g // (SUB // 16)
        t = g % (SUB // 16)
        ids = iota16 + (wid * EPT + g * 16)
        old = idxd_v[r, pl.ds(t * 16, 16)]
        spread = (ids % (NPAD - N)) + N
        idxd_v[r, pl.ds(t * 16, 16)] = jnp.where(ids >= E, spread, old)
        return carry
    lax.fori_loop(0, EPT // 16, body, 0, unroll=False)


def _sc_rows_body(pei_h, a_h, xpad_h, z48_h, z1_h, ones_h,
                  out_h, cnt_h,
                  idxs_v, idxd_v, a_v, rows_v, ones_v,
                  acc_sh, cnt_sh, semg, sems):
    c = lax.axis_index("c")
    s = lax.axis_index("s")
    pltpu.sync_copy(z48_h.at[pl.ds(s * ROWS_PER, ROWS_PER)],
                    acc_sh.at[pl.ds(s * ROWS_PER, ROWS_PER)])
    pltpu.sync_copy(z1_h.at[pl.ds(s * ROWS_PER, ROWS_PER)],
                    cnt_sh.at[pl.ds(s * ROWS_PER, ROWS_PER)])
    wid = s * NC + c
    pltpu.sync_copy(pei_h.at[0, pl.ds(wid * NSB, NSB)], idxs_v)
    pltpu.sync_copy(pei_h.at[1, pl.ds(wid * NSB, NSB)], idxd_v)
    pltpu.sync_copy(a_h.at[pl.ds(wid * EPT, EPT)], a_v)
    pltpu.sync_copy(ones_h.at[pl.ds(wid * EPT, EPT)], ones_v)
    _sanitize_dst(idxd_v, wid)
    plsc.subcore_barrier()

    def fire_gathers(k, b):
        return [pltpu.async_copy(xpad_h.at[idxs_v.at[k * CHSB + j]],
                                 rows_v.at[b, pl.ds(j * SUB, SUB)], semg)
                for j in range(CHSB)]

    def fire_scatters(k, b):
        ds = []
        for j in range(CHSB):
            ds.append(pltpu.async_copy(rows_v.at[b, pl.ds(j * SUB, SUB)],
                                       acc_sh.at[idxd_v.at[k * CHSB + j]],
                                       sems, add=True))
            ds.append(pltpu.async_copy(
                ones_v.at[pl.ds((k * CHSB + j) * SUB, SUB)],
                cnt_sh.at[idxd_v.at[k * CHSB + j]], sems, add=True))
        return ds

    def scale(k, b):
        def body(g, carry):
            av16 = a_v[pl.ds(k * CH + g * 16, 16)]
            for u in range(16):
                i = g * 16 + u
                av = av16[u]
                for t in range(F // 16):
                    sl = pl.ds(t * 16, 16)
                    rows_v[b, i, sl] = rows_v[b, i, sl] * av
            return carry
        lax.fori_loop(0, CHB // 16, body, 0, unroll=False)

    nch = EPT // CH
    gd = {0: fire_gathers(0, 0)}
    sd = {}
    for k in range(nch):
        b = k % 2
        for d in gd.pop(k % 2):
            d.wait()
        if k + 1 < nch:
            b2 = (k + 1) % 2
            for d in sd.pop(b2, []):
                d.wait()
            gd[b2] = fire_gathers(k + 1, b2)
        scale(k, b)
        sd[b] = fire_scatters(k, b)
    for ds in sd.values():
        for d in ds:
            d.wait()
    plsc.subcore_barrier()
    pltpu.sync_copy(acc_sh.at[pl.ds(s * ROWS_PER, ROWS_PER)],
                    out_h.at[c, pl.ds(s * ROWS_PER, ROWS_PER)])
    pltpu.sync_copy(cnt_sh.at[pl.ds(s * ROWS_PER, ROWS_PER)],
                    cnt_h.at[c, 0, pl.ds(s * ROWS_PER, ROWS_PER)])


_sc_rows = pl.kernel(
    _sc_rows_body,
    out_type=[
        jax.ShapeDtypeStruct((NC, NPAD, F), jnp.float32),
        jax.ShapeDtypeStruct((NC, 1, NPAD), jnp.float32),
    ],
    mesh=_MESH,
    compiler_params=pltpu.CompilerParams(use_tc_tiling_on_sc=False,
                                         needs_layout_passes=False),
    scratch_types=[
        pltpu.VMEM((NSB, SUB), jnp.int32),
        pltpu.VMEM((NSB, SUB), jnp.int32),
        pltpu.VMEM((EPT,), jnp.float32),
        pltpu.VMEM((2, CHB, F), jnp.float32),
        pltpu.VMEM((EPT,), jnp.float32),
        pltpu.VMEM_SHARED((NPAD, F), jnp.float32),
        pltpu.VMEM_SHARED((NPAD,), jnp.float32),
        pltpu.SemaphoreType.DMA,
        pltpu.SemaphoreType.DMA,
    ],
)


def _sc_scalar_body(pei_h, a_h, y_h, z1_h,
                    p_h,
                    y_v, srcs_v, idxd_v, a_v, vals_v, acc_sh, sem):
    c = lax.axis_index("c")
    s = lax.axis_index("s")
    pltpu.sync_copy(z1_h.at[pl.ds(s * ROWS_PER, ROWS_PER)],
                    acc_sh.at[pl.ds(s * ROWS_PER, ROWS_PER)])
    pltpu.sync_copy(y_h.at[0], y_v)
    wid = s * NC + c
    pltpu.sync_copy(pei_h.at[0, pl.ds(wid * NSB, NSB)], srcs_v)
    pltpu.sync_copy(pei_h.at[1, pl.ds(wid * NSB, NSB)], idxd_v)
    pltpu.sync_copy(a_h.at[pl.ds(wid * EPT, EPT)], a_v)
    _sanitize_dst(idxd_v, wid)
    plsc.subcore_barrier()

    def gath(g, carry):
        r = g // (SUB // 16)
        t = g % (SUB // 16)
        idx16 = srcs_v[r, pl.ds(t * 16, 16)]
        vals = plsc.load_gather(y_v, [idx16])
        vals_v[pl.ds(g * 16, 16)] = vals * a_v[pl.ds(g * 16, 16)]
        return carry
    lax.fori_loop(0, EPT // 16, gath, 0, unroll=False)

    descs = [pltpu.async_copy(vals_v.at[pl.ds(j * SUB, SUB)],
                              acc_sh.at[idxd_v.at[j]], sem, add=True)
             for j in range(NSB)]
    for d in descs:
        d.wait()
    plsc.subcore_barrier()
    pltpu.sync_copy(acc_sh.at[pl.ds(s * ROWS_PER, ROWS_PER)],
                    p_h.at[c, 0, pl.ds(s * ROWS_PER, ROWS_PER)])


_sc_scalar = pl.kernel(
    _sc_scalar_body,
    out_type=jax.ShapeDtypeStruct((NC, 1, NPAD), jnp.float32),
    mesh=_MESH,
    compiler_params=pltpu.CompilerParams(needs_layout_passes=False),
    scratch_types=[
        pltpu.VMEM((NPAD,), jnp.float32),
        pltpu.VMEM((NSB, SUB), jnp.int32),
        pltpu.VMEM((NSB, SUB), jnp.int32),
        pltpu.VMEM((EPT,), jnp.float32),
        pltpu.VMEM((EPT,), jnp.float32),
        pltpu.VMEM_SHARED((NPAD,), jnp.float32),
        pltpu.SemaphoreType.DMA,
    ],
)


LB = 2048
NLCH = NPAD // LB


def _dot(a, b):
    return jax.lax.dot(a, b, precision=jax.lax.Precision.HIGHEST,
                       preferred_element_type=jnp.float32)


def _dott(a, b):
    return lax.dot_general(a, b, (((1,), (1,)), ((), ())),
                           precision=jax.lax.Precision.HIGHEST,
                           preferred_element_type=jnp.float32)


def _lane_mask(l0):
    lanes = lax.broadcasted_iota(jnp.int32, (1, LB), 1) + l0
    return (lanes < N).astype(jnp.float32)


def _tc1_body(pp, cp, xp, w1t, root1t, bias1c, g1c, b1c, we2r,
              x1T_o, y1_o, cntc_o, h_v, st_v):
    p = pl.program_id(0)
    i = pl.program_id(1)
    l0 = i * LB

    @pl.when(p == 0)
    def _():
        @pl.when(i == 0)
        def _():
            st_v[...] = jnp.zeros((F, 8), jnp.float32)
        R1t = jnp.maximum(w1t[...], 0.0)
        cnt = jnp.maximum(cp[0, :, :] + cp[1, :, :], 1.0)
        P = pp[0] + pp[1]
        hT = _dott(R1t, P) / cnt + _dott(root1t[...], xp[...]) + bias1c[...]
        h_v[:, pl.ds(l0, LB)] = hT
        hm = hT * _lane_mask(l0)
        st_v[:, 0:1] = st_v[:, 0:1] + jnp.sum(hm, 1, keepdims=True)
        st_v[:, 1:2] = st_v[:, 1:2] + jnp.sum(hm * hm, 1, keepdims=True)

    @pl.when(p == 1)
    def _():
        m = st_v[:, 0:1] / N
        v = st_v[:, 1:2] / N - m * m
        scale = g1c[...] / jnp.sqrt(v + EPS)
        off = b1c[...] - m * scale
        x1T = jax.nn.sigmoid(h_v[:, pl.ds(l0, LB)] * scale + off)
        x1T_o[...] = x1T
        y1_o[...] = _dot(jnp.maximum(we2r[...], 0.0), x1T)
        cntc_o[...] = jnp.maximum(cp[0, :, :] + cp[1, :, :], 1.0)


_tc1 = pl.pallas_call(
    _tc1_body,
    grid=(2, NLCH),
    in_specs=[
        pl.BlockSpec((NC, LB, F), lambda p, i: (0, i * (1 - p), 0)),
        pl.BlockSpec((NC, 1, LB), lambda p, i: (0, 0, i)),
        pl.BlockSpec((LB, F), lambda p, i: (i * (1 - p), 0)),
        pl.BlockSpec((F, F), lambda p, i: (0, 0)),
        pl.BlockSpec((F, F), lambda p, i: (0, 0)),
        pl.BlockSpec((F, 1), lambda p, i: (0, 0)),
        pl.BlockSpec((F, 1), lambda p, i: (0, 0)),
        pl.BlockSpec((F, 1), lambda p, i: (0, 0)),
        pl.BlockSpec((1, F), lambda p, i: (0, 0)),
    ],
    out_specs=[
        pl.BlockSpec((F, LB), lambda p, i: (0, i * p)),
        pl.BlockSpec((1, LB), lambda p, i: (0, i * p)),
        pl.BlockSpec((1, LB), lambda p, i: (0, i * p)),
    ],
    out_shape=[
        jax.ShapeDtypeStruct((F, NPAD), jnp.float32),
        jax.ShapeDtypeStruct((1, NPAD), jnp.float32),
        jax.ShapeDtypeStruct((1, NPAD), jnp.float32),
    ],
    scratch_shapes=[pltpu.VMEM((F, NPAD), jnp.float32),
                    pltpu.VMEM((F, 8), jnp.float32)],
)


def _tc2_body(p2p, cntc, x1T, root2t, bias2, g2, b2, x2_o):
    p2 = p2p[0, :, :] + p2p[1, :, :]
    h = p2 / cntc[...] + _dot(root2t[...], x1T[...]) + bias2[...]
    mask = (lax.broadcasted_iota(jnp.int32, (1, NPAD), 1) < N
            ).astype(jnp.float32)
    hm = h * mask
    m = jnp.sum(hm, 1, keepdims=True) / N
    v = jnp.sum(hm * hm, 1, keepdims=True) / N - m * m
    x2_o[...] = jax.nn.sigmoid(
        g2[...] * (h - m) / jnp.sqrt(v + EPS) + b2[...])


_tc2 = pl.pallas_call(
    _tc2_body,
    out_shape=jax.ShapeDtypeStruct((1, NPAD), jnp.float32),
)


def _tc3_body(p3p, cntc, x1T, x2, we3c, root3c, bias3c, g3c, b3c, out_o):
    R3c = jnp.maximum(we3c[...], 0.0)
    RTc = root3c[...]
    z = jnp.zeros((F, 1), jnp.float32)

    def hchunk(l0):
        lsl = pl.ds(l0, LB)
        p3n = (p3p[0, :, lsl] + p3p[1, :, lsl]) / cntc[:, lsl]
        return lsl, R3c * p3n + RTc * x2[:, lsl] + bias3c[...]

    def pass1(i, carry):
        sh, sq = carry
        l0 = i * LB
        _, hT = hchunk(l0)
        hm = hT * _lane_mask(l0)
        return (sh + jnp.sum(hm, 1, keepdims=True),
                sq + jnp.sum(hm * hm, 1, keepdims=True))

    sh, sq = lax.fori_loop(0, NLCH, pass1, (z, z))
    m = sh / N
    v = sq / N - m * m
    scale = g3c[...] / jnp.sqrt(v + EPS)
    off = b3c[...] - m * scale

    def pass2(i, carry):
        lsl, hT = hchunk(i * LB)
        x3aT = jax.nn.sigmoid(hT * scale + off)
        out_o[:, lsl] = (x3aT + x1T[:, lsl]) * 0.5
        return carry

    lax.fori_loop(0, NLCH, pass2, 0)


_tc3 = pl.pallas_call(
    _tc3_body,
    out_shape=jax.ShapeDtypeStruct((F, NPAD), jnp.float32),
)


def _pad2(w, rows, cols):
    return jnp.pad(w, ((0, rows - w.shape[0]), (0, cols - w.shape[1])))


def kernel(x, pos_edge_index, edge_attr,
           We1, be1, root1, bias1, g1, b1,
           We2, be2, root2, bias2, g2, b2,
           We3, be3, root3, bias3, g3, b3):
    f32 = jnp.float32
    del be1, be2, be3

    pad_e = EPAD - E
    pei = jnp.pad(pos_edge_index.astype(jnp.int32),
                  ((0, 0), (0, pad_e))).reshape(2, EPAD // SUB, SUB)
    ap = jnp.pad(edge_attr.reshape(E).astype(f32), (0, pad_e))
    onesp = jnp.pad(jnp.ones((E,), f32), (0, pad_e))

    xpad = jnp.pad(x, ((0, NPAD - N), (0, F - 35)))
    z48 = jnp.zeros((NPAD, F), f32)
    z1 = jnp.zeros((NPAD,), f32)

    w1t = _pad2(We1.reshape(35, 35).T, F, F)
    root1t = _pad2(root1.T, F, F)
    bias1c = jnp.pad(bias1, (0, F - 35)).reshape(F, 1)
    g1c = jnp.pad(g1, (0, F - 35)).reshape(F, 1)
    b1c = jnp.pad(b1, (0, F - 35)).reshape(F, 1)
    we2r = _pad2(We2, 1, F)
    root2t = _pad2(root2.reshape(1, 35), 1, F)
    bias2r = bias2.reshape(1, 1)
    g2r = g2.reshape(1, 1)
    b2r = b2.reshape(1, 1)
    we3c = _pad2(We3.reshape(35, 1), F, 1)
    root3c = _pad2(root3.reshape(35, 1), F, 1)
    bias3c = jnp.pad(bias3, (0, F - 35)).reshape(F, 1)
    g3c = jnp.pad(g3, (0, F - 35)).reshape(F, 1)
    b3c = jnp.pad(b3, (0, F - 35)).reshape(F, 1)

    P1p, cntp = _sc_rows(pei, ap, xpad, z48, z1, onesp)
    x1T, y1, cntc = _tc1(P1p, cntp, xpad,
                         w1t, root1t, bias1c, g1c, b1c, we2r)
    p2p = _sc_scalar(pei, ap, y1, z1)
    x2 = _tc2(p2p, cntc, x1T, root2t, bias2r, g2r, b2r)
    p3p = _sc_scalar(pei, ap, x2, z1)
    outT = _tc3(p3p, cntc, x1T, x2, we3c, root3c, bias3c, g3c, b3c)
    return outT[:35, :N].T


# speedup vs baseline: 1.4863x; 1.4863x over previous
import functools

import jax
import jax.numpy as jnp
from jax import lax
from jax.experimental import pallas as pl
from jax.experimental.pallas import tpu as pltpu
from jax.experimental.pallas import tpu_sc as plsc

N = 10000
E = 160000
EPS = 1e-3
F = 48
NPAD = 10240
NC, NS = 2, 16
NW = NC * NS
EPT = 5120
EPAD = EPT * NW
SUB = 128
NSB = EPT // SUB
CH = 512
CHSB = CH // SUB
CHB = CH
ROWS_PER = NPAD // NS

_MESH = plsc.VectorSubcoreMesh(core_axis_name="c", subcore_axis_name="s")


def _sanitize_idx(idxd_v, wid):
    iota16 = lax.iota(jnp.int32, 16)

    def body(g, carry):
        r = g // (SUB // 16)
        t = g % (SUB // 16)
        ids = iota16 + (wid * EPT + g * 16)
        old = idxd_v[r, pl.ds(t * 16, 16)]
        spread = (ids % (NPAD - N)) + N
        idxd_v[r, pl.ds(t * 16, 16)] = jnp.where(ids >= E, spread, old)
        return carry
    lax.fori_loop(0, EPT // 16, body, 0, unroll=False)


def _sc_rows_body(pei_h, a_h, xpad_h, z48_h, z1_h, ones_h,
                  out_h, cnt_h,
                  idxs_v, idxd_v, a_v, rows_v, ones_v,
                  acc_sh, cnt_sh, semg, sems):
    c = lax.axis_index("c")
    s = lax.axis_index("s")
    pltpu.sync_copy(z48_h.at[pl.ds(s * ROWS_PER, ROWS_PER)],
                    acc_sh.at[pl.ds(s * ROWS_PER, ROWS_PER)])
    pltpu.sync_copy(z1_h.at[pl.ds(s * ROWS_PER, ROWS_PER)],
                    cnt_sh.at[pl.ds(s * ROWS_PER, ROWS_PER)])
    wid = s * NC + c
    pltpu.sync_copy(pei_h.at[0, pl.ds(wid * NSB, NSB)], idxs_v)
    pltpu.sync_copy(pei_h.at[1, pl.ds(wid * NSB, NSB)], idxd_v)
    pltpu.sync_copy(a_h.at[pl.ds(wid * EPT, EPT)], a_v)
    pltpu.sync_copy(ones_h.at[pl.ds(wid * EPT, EPT)], ones_v)
    _sanitize_idx(idxs_v, wid)
    _sanitize_idx(idxd_v, wid)
    plsc.subcore_barrier()

    def fire_gathers(k, b):
        return [pltpu.async_copy(xpad_h.at[idxs_v.at[k * CHSB + j]],
                                 rows_v.at[b, pl.ds(j * SUB, SUB)], semg)
                for j in range(CHSB)]

    def fire_scatters(k, b):
        ds = []
        for j in range(CHSB):
            ds.append(pltpu.async_copy(rows_v.at[b, pl.ds(j * SUB, SUB)],
                                       acc_sh.at[idxd_v.at[k * CHSB + j]],
                                       sems, add=True))
            ds.append(pltpu.async_copy(
                ones_v.at[pl.ds((k * CHSB + j) * SUB, SUB)],
                cnt_sh.at[idxd_v.at[k * CHSB + j]], sems, add=True))
        return ds

    def scale(k, b):
        def body(g, carry):
            av16 = a_v[pl.ds(k * CH + g * 16, 16)]
            for u in range(16):
                i = g * 16 + u
                av = av16[u]
                for t in range(F // 16):
                    sl = pl.ds(t * 16, 16)
                    rows_v[b, i, sl] = rows_v[b, i, sl] * av
            return carry
        lax.fori_loop(0, CHB // 16, body, 0, unroll=False)

    nch = EPT // CH
    gd = {0: fire_gathers(0, 0)}
    sd = {}
    for k in range(nch):
        b = k % 2
        for d in gd.pop(k % 2):
            d.wait()
        if k + 1 < nch:
            b2 = (k + 1) % 2
            for d in sd.pop(b2, []):
                d.wait()
            gd[b2] = fire_gathers(k + 1, b2)
        scale(k, b)
        sd[b] = fire_scatters(k, b)
    for ds in sd.values():
        for d in ds:
            d.wait()
    plsc.subcore_barrier()
    pltpu.sync_copy(acc_sh.at[pl.ds(s * ROWS_PER, ROWS_PER)],
                    out_h.at[c, pl.ds(s * ROWS_PER, ROWS_PER)])
    pltpu.sync_copy(cnt_sh.at[pl.ds(s * ROWS_PER, ROWS_PER)],
                    cnt_h.at[c, 0, pl.ds(s * ROWS_PER, ROWS_PER)])


_sc_rows = pl.kernel(
    _sc_rows_body,
    out_type=[
        jax.ShapeDtypeStruct((NC, NPAD, F), jnp.float32),
        jax.ShapeDtypeStruct((NC, 1, NPAD), jnp.float32),
    ],
    mesh=_MESH,
    compiler_params=pltpu.CompilerParams(use_tc_tiling_on_sc=False,
                                         needs_layout_passes=False),
    scratch_types=[
        pltpu.VMEM((NSB, SUB), jnp.int32),
        pltpu.VMEM((NSB, SUB), jnp.int32),
        pltpu.VMEM((EPT,), jnp.float32),
        pltpu.VMEM((2, CHB, F), jnp.float32),
        pltpu.VMEM((EPT,), jnp.float32),
        pltpu.VMEM_SHARED((NPAD, F), jnp.float32),
        pltpu.VMEM_SHARED((NPAD,), jnp.float32),
        pltpu.SemaphoreType.DMA,
        pltpu.SemaphoreType.DMA,
    ],
)


def _sc_scalar_body(pei_h, a_h, y_h, z1_h,
                    p_h,
                    y_v, srcs_v, idxd_v, a_v, vals_v, acc_sh, sem):
    c = lax.axis_index("c")
    s = lax.axis_index("s")
    pltpu.sync_copy(z1_h.at[pl.ds(s * ROWS_PER, ROWS_PER)],
                    acc_sh.at[pl.ds(s * ROWS_PER, ROWS_PER)])
    pltpu.sync_copy(y_h.at[0], y_v)
    wid = s * NC + c
    pltpu.sync_copy(pei_h.at[0, pl.ds(wid * NSB, NSB)], srcs_v)
    pltpu.sync_copy(pei_h.at[1, pl.ds(wid * NSB, NSB)], idxd_v)
    pltpu.sync_copy(a_h.at[pl.ds(wid * EPT, EPT)], a_v)
    _sanitize_idx(srcs_v, wid)
    _sanitize_idx(idxd_v, wid)
    plsc.subcore_barrier()

    def gath(g, carry):
        r = g // (SUB // 16)
        t = g % (SUB // 16)
        idx16 = srcs_v[r, pl.ds(t * 16, 16)]
        vals = plsc.load_gather(y_v, [idx16])
        vals_v[pl.ds(g * 16, 16)] = vals * a_v[pl.ds(g * 16, 16)]
        return carry
    lax.fori_loop(0, EPT // 16, gath, 0, unroll=False)

    descs = [pltpu.async_copy(vals_v.at[pl.ds(j * SUB, SUB)],
                              acc_sh.at[idxd_v.at[j]], sem, add=True)
             for j in range(NSB)]
    for d in descs:
        d.wait()
    plsc.subcore_barrier()
    pltpu.sync_copy(acc_sh.at[pl.ds(s * ROWS_PER, ROWS_PER)],
                    p_h.at[c, 0, pl.ds(s * ROWS_PER, ROWS_PER)])


_sc_scalar = pl.kernel(
    _sc_scalar_body,
    out_type=jax.ShapeDtypeStruct((NC, 1, NPAD), jnp.float32),
    mesh=_MESH,
    compiler_params=pltpu.CompilerParams(needs_layout_passes=False),
    scratch_types=[
        pltpu.VMEM((NPAD,), jnp.float32),
        pltpu.VMEM((NSB, SUB), jnp.int32),
        pltpu.VMEM((NSB, SUB), jnp.int32),
        pltpu.VMEM((EPT,), jnp.float32),
        pltpu.VMEM((EPT,), jnp.float32),
        pltpu.VMEM_SHARED((NPAD,), jnp.float32),
        pltpu.SemaphoreType.DMA,
    ],
)


LB = 2048
NLCH = NPAD // LB


def _dot(a, b):
    return jax.lax.dot(a, b, precision=jax.lax.Precision.HIGHEST,
                       preferred_element_type=jnp.float32)


def _dott(a, b):
    return lax.dot_general(a, b, (((1,), (1,)), ((), ())),
                           precision=jax.lax.Precision.HIGHEST,
                           preferred_element_type=jnp.float32)


def _lane_mask(l0):
    lanes = lax.broadcasted_iota(jnp.int32, (1, LB), 1) + l0
    return (lanes < N).astype(jnp.float32)


def _tc1_body(pp, cp, xp, w1t, root1t, bias1c, g1c, b1c, we2r,
              x1T_o, y1_o, cntc_o, h_v, st_v):
    p = pl.program_id(0)
    i = pl.program_id(1)
    l0 = i * LB

    @pl.when(p == 0)
    def _():
        @pl.when(i == 0)
        def _():
            st_v[...] = jnp.zeros((F, 8), jnp.float32)
        R1t = jnp.maximum(w1t[...], 0.0)
        cnt = jnp.maximum(cp[0, :, :] + cp[1, :, :], 1.0)
        P = pp[0] + pp[1]
        hT = _dott(R1t, P) / cnt + _dott(root1t[...], xp[...]) + bias1c[...]
        h_v[:, pl.ds(l0, LB)] = hT
        hm = hT * _lane_mask(l0)
        st_v[:, 0:1] = st_v[:, 0:1] + jnp.sum(hm, 1, keepdims=True)
        st_v[:, 1:2] = st_v[:, 1:2] + jnp.sum(hm * hm, 1, keepdims=True)

    @pl.when(p == 1)
    def _():
        m = st_v[:, 0:1] / N
        v = st_v[:, 1:2] / N - m * m
        scale = g1c[...] / jnp.sqrt(v + EPS)
        off = b1c[...] - m * scale
        x1T = jax.nn.sigmoid(h_v[:, pl.ds(l0, LB)] * scale + off)
        x1T_o[...] = x1T
        y1_o[...] = _dot(jnp.maximum(we2r[...], 0.0), x1T)
        cntc_o[...] = jnp.maximum(cp[0, :, :] + cp[1, :, :], 1.0)


_tc1 = pl.pallas_call(
    _tc1_body,
    grid=(2, NLCH),
    in_specs=[
        pl.BlockSpec((NC, LB, F), lambda p, i: (0, i * (1 - p), 0)),
        pl.BlockSpec((NC, 1, LB), lambda p, i: (0, 0, i)),
        pl.BlockSpec((LB, F), lambda p, i: (i * (1 - p), 0)),
        pl.BlockSpec((F, F), lambda p, i: (0, 0)),
        pl.BlockSpec((F, F), lambda p, i: (0, 0)),
        pl.BlockSpec((F, 1), lambda p, i: (0, 0)),
        pl.BlockSpec((F, 1), lambda p, i: (0, 0)),
        pl.BlockSpec((F, 1), lambda p, i: (0, 0)),
        pl.BlockSpec((1, F), lambda p, i: (0, 0)),
    ],
    out_specs=[
        pl.BlockSpec((F, LB), lambda p, i: (0, i * p)),
        pl.BlockSpec((1, LB), lambda p, i: (0, i * p)),
        pl.BlockSpec((1, LB), lambda p, i: (0, i * p)),
    ],
    out_shape=[
        jax.ShapeDtypeStruct((F, NPAD), jnp.float32),
        jax.ShapeDtypeStruct((1, NPAD), jnp.float32),
        jax.ShapeDtypeStruct((1, NPAD), jnp.float32),
    ],
    scratch_shapes=[pltpu.VMEM((F, NPAD), jnp.float32),
                    pltpu.VMEM((F, 8), jnp.float32)],
)


def _tc2_body(p2p, cntc, x1T, root2t, bias2, g2, b2, x2_o):
    p2 = p2p[0, :, :] + p2p[1, :, :]
    h = p2 / cntc[...] + _dot(root2t[...], x1T[...]) + bias2[...]
    mask = (lax.broadcasted_iota(jnp.int32, (1, NPAD), 1) < N
            ).astype(jnp.float32)
    hm = h * mask
    m = jnp.sum(hm, 1, keepdims=True) / N
    v = jnp.sum(hm * hm, 1, keepdims=True) / N - m * m
    x2_o[...] = jax.nn.sigmoid(
        g2[...] * (h - m) / jnp.sqrt(v + EPS) + b2[...])


_tc2 = pl.pallas_call(
    _tc2_body,
    out_shape=jax.ShapeDtypeStruct((1, NPAD), jnp.float32),
)


def _tc3_body(p3p, cntc, x1T, x2, we3c, root3c, bias3c, g3c, b3c, out_o):
    R3c = jnp.maximum(we3c[...], 0.0)
    RTc = root3c[...]
    z = jnp.zeros((F, 1), jnp.float32)

    def hchunk(l0):
        lsl = pl.ds(l0, LB)
        p3n = (p3p[0, :, lsl] + p3p[1, :, lsl]) / cntc[:, lsl]
        return lsl, R3c * p3n + RTc * x2[:, lsl] + bias3c[...]

    def pass1(i, carry):
        sh, sq = carry
        l0 = i * LB
        _, hT = hchunk(l0)
        hm = hT * _lane_mask(l0)
        return (sh + jnp.sum(hm, 1, keepdims=True),
                sq + jnp.sum(hm * hm, 1, keepdims=True))

    sh, sq = lax.fori_loop(0, NLCH, pass1, (z, z))
    m = sh / N
    v = sq / N - m * m
    scale = g3c[...] / jnp.sqrt(v + EPS)
    off = b3c[...] - m * scale

    def pass2(i, carry):
        lsl, hT = hchunk(i * LB)
        x3aT = jax.nn.sigmoid(hT * scale + off)
        out_o[:, lsl] = (x3aT + x1T[:, lsl]) * 0.5
        return carry

    lax.fori_loop(0, NLCH, pass2, 0)


_tc3 = pl.pallas_call(
    _tc3_body,
    out_shape=jax.ShapeDtypeStruct((F, NPAD), jnp.float32),
)


def _pad2(w, rows, cols):
    return jnp.pad(w, ((0, rows - w.shape[0]), (0, cols - w.shape[1])))


def kernel(x, pos_edge_index, edge_attr,
           We1, be1, root1, bias1, g1, b1,
           We2, be2, root2, bias2, g2, b2,
           We3, be3, root3, bias3, g3, b3):
    f32 = jnp.float32
    del be1, be2, be3

    pad_e = EPAD - E
    pei = jnp.pad(pos_edge_index.astype(jnp.int32),
                  ((0, 0), (0, pad_e))).reshape(2, EPAD // SUB, SUB)
    ap = jnp.pad(edge_attr.reshape(E).astype(f32), (0, pad_e))
    onesp = jnp.pad(jnp.ones((E,), f32), (0, pad_e))

    xpad = jnp.pad(x, ((0, NPAD - N), (0, F - 35)))
    z48 = jnp.zeros((NPAD, F), f32)
    z1 = jnp.zeros((NPAD,), f32)

    w1t = _pad2(We1.reshape(35, 35).T, F, F)
    root1t = _pad2(root1.T, F, F)
    bias1c = jnp.pad(bias1, (0, F - 35)).reshape(F, 1)
    g1c = jnp.pad(g1, (0, F - 35)).reshape(F, 1)
    b1c = jnp.pad(b1, (0, F - 35)).reshape(F, 1)
    we2r = _pad2(We2, 1, F)
    root2t = _pad2(root2.reshape(1, 35), 1, F)
    bias2r = bias2.reshape(1, 1)
    g2r = g2.reshape(1, 1)
    b2r = b2.reshape(1, 1)
    we3c = _pad2(We3.reshape(35, 1), F, 1)
    root3c = _pad2(root3.reshape(35, 1), F, 1)
    bias3c = jnp.pad(bias3, (0, F - 35)).reshape(F, 1)
    g3c = jnp.pad(g3, (0, F - 35)).reshape(F, 1)
    b3c = jnp.pad(b3, (0, F - 35)).reshape(F, 1)

    P1p, cntp = _sc_rows(pei, ap, xpad, z48, z1, onesp)
    x1T, y1, cntc = _tc1(P1p, cntp, xpad,
                         w1t, root1t, bias1c, g1c, b1c, we2r)
    p2p = _sc_scalar(pei, ap, y1, z1)
    x2 = _tc2(p2p, cntc, x1T, root2t, bias2r, g2r, b2r)
    p3p = _sc_scalar(pei, ap, x2, z1)
    outT = _tc3(p3p, cntc, x1T, x2, we3c, root3c, bias3c, g3c, b3c)
    return outT[:35, :N].T


# speedup vs baseline: 1.5607x; 1.0500x over previous
import functools

import jax
import jax.numpy as jnp
from jax import lax
from jax.experimental import pallas as pl
from jax.experimental.pallas import tpu as pltpu
from jax.experimental.pallas import tpu_sc as plsc

N = 10000
E = 160000
EPS = 1e-3
F = 48
NPAD = 10240
NC, NS = 2, 16
NW = NC * NS
EPT = 5120
EPAD = EPT * NW
SUB = 128
NSB = EPT // SUB
CH = 512
CHSB = CH // SUB
CHB = CH
ROWS_PER = NPAD // NS

_MESH = plsc.VectorSubcoreMesh(core_axis_name="c", subcore_axis_name="s")


def _sc_rows_body(pei_h, a_h, xpad_h, z48_h, z1_h, ones_h,
                  out_h, cnt_h,
                  idxs_v, idxd_v, a_v, rows_v, ones_v,
                  acc_sh, cnt_sh, semg, sems):
    c = lax.axis_index("c")
    s = lax.axis_index("s")
    pltpu.sync_copy(z48_h.at[pl.ds(s * ROWS_PER, ROWS_PER)],
                    acc_sh.at[pl.ds(s * ROWS_PER, ROWS_PER)])
    pltpu.sync_copy(z1_h.at[pl.ds(s * ROWS_PER, ROWS_PER)],
                    cnt_sh.at[pl.ds(s * ROWS_PER, ROWS_PER)])
    wid = s * NC + c
    pltpu.sync_copy(pei_h.at[0, pl.ds(wid * NSB, NSB)], idxs_v)
    pltpu.sync_copy(pei_h.at[1, pl.ds(wid * NSB, NSB)], idxd_v)
    pltpu.sync_copy(a_h.at[pl.ds(wid * EPT, EPT)], a_v)
    pltpu.sync_copy(ones_h.at[pl.ds(wid * EPT, EPT)], ones_v)
    plsc.subcore_barrier()

    def fire_gathers(k, b):
        return [pltpu.async_copy(xpad_h.at[idxs_v.at[k * CHSB + j]],
                                 rows_v.at[b, pl.ds(j * SUB, SUB)], semg)
                for j in range(CHSB)]

    def fire_scatters(k, b):
        ds = []
        for j in range(CHSB):
            ds.append(pltpu.async_copy(rows_v.at[b, pl.ds(j * SUB, SUB)],
                                       acc_sh.at[idxd_v.at[k * CHSB + j]],
                                       sems, add=True))
            ds.append(pltpu.async_copy(
                ones_v.at[pl.ds((k * CHSB + j) * SUB, SUB)],
                cnt_sh.at[idxd_v.at[k * CHSB + j]], sems, add=True))
        return ds

    def scale(k, b):
        def body(g, carry):
            av16 = a_v[pl.ds(k * CH + g * 16, 16)]
            for u in range(16):
                i = g * 16 + u
                av = av16[u]
                for t in range(F // 16):
                    sl = pl.ds(t * 16, 16)
                    rows_v[b, i, sl] = rows_v[b, i, sl] * av
            return carry
        lax.fori_loop(0, CHB // 16, body, 0, unroll=False)

    nch = EPT // CH
    gd = {0: fire_gathers(0, 0)}
    sd = {}
    for k in range(nch):
        b = k % 2
        for d in gd.pop(k % 2):
            d.wait()
        if k + 1 < nch:
            b2 = (k + 1) % 2
            for d in sd.pop(b2, []):
                d.wait()
            gd[b2] = fire_gathers(k + 1, b2)
        scale(k, b)
        sd[b] = fire_scatters(k, b)
    for ds in sd.values():
        for d in ds:
            d.wait()
    plsc.subcore_barrier()
    pltpu.sync_copy(acc_sh.at[pl.ds(s * ROWS_PER, ROWS_PER)],
                    out_h.at[c, pl.ds(s * ROWS_PER, ROWS_PER)])
    pltpu.sync_copy(cnt_sh.at[pl.ds(s * ROWS_PER, ROWS_PER)],
                    cnt_h.at[c, 0, pl.ds(s * ROWS_PER, ROWS_PER)])


_sc_rows = pl.kernel(
    _sc_rows_body,
    out_type=[
        jax.ShapeDtypeStruct((NC, NPAD, F), jnp.float32),
        jax.ShapeDtypeStruct((NC, 1, NPAD), jnp.float32),
    ],
    mesh=_MESH,
    compiler_params=pltpu.CompilerParams(use_tc_tiling_on_sc=False,
                                         needs_layout_passes=False),
    scratch_types=[
        pltpu.VMEM((NSB, SUB), jnp.int32),
        pltpu.VMEM((NSB, SUB), jnp.int32),
        pltpu.VMEM((EPT,), jnp.float32),
        pltpu.VMEM((2, CHB, F), jnp.float32),
        pltpu.VMEM((EPT,), jnp.float32),
        pltpu.VMEM_SHARED((NPAD, F), jnp.float32),
        pltpu.VMEM_SHARED((NPAD,), jnp.float32),
        pltpu.SemaphoreType.DMA,
        pltpu.SemaphoreType.DMA,
    ],
)


def _sc_scalar_body(pei_h, a_h, y_h, z1_h,
                    p_h,
                    y_v, srcs_v, idxd_v, a_v, vals_v, acc_sh, sem):
    c = lax.axis_index("c")
    s = lax.axis_index("s")
    pltpu.sync_copy(z1_h.at[pl.ds(s * ROWS_PER, ROWS_PER)],
                    acc_sh.at[pl.ds(s * ROWS_PER, ROWS_PER)])
    pltpu.sync_copy(y_h.at[0], y_v)
    wid = s * NC + c
    pltpu.sync_copy(pei_h.at[0, pl.ds(wid * NSB, NSB)], srcs_v)
    pltpu.sync_copy(pei_h.at[1, pl.ds(wid * NSB, NSB)], idxd_v)
    pltpu.sync_copy(a_h.at[pl.ds(wid * EPT, EPT)], a_v)
    plsc.subcore_barrier()

    def gath(g, carry):
        r = g // (SUB // 16)
        t = g % (SUB // 16)
        idx16 = srcs_v[r, pl.ds(t * 16, 16)]
        vals = plsc.load_gather(y_v, [idx16])
        vals_v[pl.ds(g * 16, 16)] = vals * a_v[pl.ds(g * 16, 16)]
        return carry
    lax.fori_loop(0, EPT // 16, gath, 0, unroll=False)

    descs = [pltpu.async_copy(vals_v.at[pl.ds(j * SUB, SUB)],
                              acc_sh.at[idxd_v.at[j]], sem, add=True)
             for j in range(NSB)]
    for d in descs:
        d.wait()
    plsc.subcore_barrier()
    pltpu.sync_copy(acc_sh.at[pl.ds(s * ROWS_PER, ROWS_PER)],
                    p_h.at[c, 0, pl.ds(s * ROWS_PER, ROWS_PER)])


_sc_scalar = pl.kernel(
    _sc_scalar_body,
    out_type=jax.ShapeDtypeStruct((NC, 1, NPAD), jnp.float32),
    mesh=_MESH,
    compiler_params=pltpu.CompilerParams(needs_layout_passes=False),
    scratch_types=[
        pltpu.VMEM((NPAD,), jnp.float32),
        pltpu.VMEM((NSB, SUB), jnp.int32),
        pltpu.VMEM((NSB, SUB), jnp.int32),
        pltpu.VMEM((EPT,), jnp.float32),
        pltpu.VMEM((EPT,), jnp.float32),
        pltpu.VMEM_SHARED((NPAD,), jnp.float32),
        pltpu.SemaphoreType.DMA,
    ],
)


LB = 2048
NLCH = NPAD // LB


def _dot(a, b):
    return jax.lax.dot(a, b, precision=jax.lax.Precision.HIGHEST,
                       preferred_element_type=jnp.float32)


def _dott(a, b):
    return lax.dot_general(a, b, (((1,), (1,)), ((), ())),
                           precision=jax.lax.Precision.HIGHEST,
                           preferred_element_type=jnp.float32)


def _lane_mask(l0):
    lanes = lax.broadcasted_iota(jnp.int32, (1, LB), 1) + l0
    return (lanes < N).astype(jnp.float32)


def _tc1_body(pp, cp, xp, w1t, root1t, bias1c, g1c, b1c, we2r,
              x1T_o, y1_o, cntc_o, h_v, st_v):
    p = pl.program_id(0)
    i = pl.program_id(1)
    l0 = i * LB

    @pl.when(p == 0)
    def _():
        @pl.when(i == 0)
        def _():
            st_v[...] = jnp.zeros((F, 8), jnp.float32)
        R1t = jnp.maximum(w1t[...], 0.0)
        cnt = jnp.maximum(cp[0, :, :] + cp[1, :, :], 1.0)
        P = pp[0] + pp[1]
        hT = _dott(R1t, P) / cnt + _dott(root1t[...], xp[...]) + bias1c[...]
        h_v[:, pl.ds(l0, LB)] = hT
        hm = hT * _lane_mask(l0)
        st_v[:, 0:1] = st_v[:, 0:1] + jnp.sum(hm, 1, keepdims=True)
        st_v[:, 1:2] = st_v[:, 1:2] + jnp.sum(hm * hm, 1, keepdims=True)

    @pl.when(p == 1)
    def _():
        m = st_v[:, 0:1] / N
        v = st_v[:, 1:2] / N - m * m
        scale = g1c[...] / jnp.sqrt(v + EPS)
        off = b1c[...] - m * scale
        x1T = jax.nn.sigmoid(h_v[:, pl.ds(l0, LB)] * scale + off)
        x1T_o[...] = x1T
        y1_o[...] = _dot(jnp.maximum(we2r[...], 0.0), x1T)
        cntc_o[...] = jnp.maximum(cp[0, :, :] + cp[1, :, :], 1.0)


_tc1 = pl.pallas_call(
    _tc1_body,
    grid=(2, NLCH),
    in_specs=[
        pl.BlockSpec((NC, LB, F), lambda p, i: (0, i * (1 - p), 0)),
        pl.BlockSpec((NC, 1, LB), lambda p, i: (0, 0, i)),
        pl.BlockSpec((LB, F), lambda p, i: (i * (1 - p), 0)),
        pl.BlockSpec((F, F), lambda p, i: (0, 0)),
        pl.BlockSpec((F, F), lambda p, i: (0, 0)),
        pl.BlockSpec((F, 1), lambda p, i: (0, 0)),
        pl.BlockSpec((F, 1), lambda p, i: (0, 0)),
        pl.BlockSpec((F, 1), lambda p, i: (0, 0)),
        pl.BlockSpec((1, F), lambda p, i: (0, 0)),
    ],
    out_specs=[
        pl.BlockSpec((F, LB), lambda p, i: (0, i * p)),
        pl.BlockSpec((1, LB), lambda p, i: (0, i * p)),
        pl.BlockSpec((1, LB), lambda p, i: (0, i * p)),
    ],
    out_shape=[
        jax.ShapeDtypeStruct((F, NPAD), jnp.float32),
        jax.ShapeDtypeStruct((1, NPAD), jnp.float32),
        jax.ShapeDtypeStruct((1, NPAD), jnp.float32),
    ],
    scratch_shapes=[pltpu.VMEM((F, NPAD), jnp.float32),
                    pltpu.VMEM((F, 8), jnp.float32)],
)


def _tc2_body(p2p, cntc, x1T, root2t, bias2, g2, b2, x2_o):
    p2 = p2p[0, :, :] + p2p[1, :, :]
    h = p2 / cntc[...] + _dot(root2t[...], x1T[...]) + bias2[...]
    mask = (lax.broadcasted_iota(jnp.int32, (1, NPAD), 1) < N
            ).astype(jnp.float32)
    hm = h * mask
    m = jnp.sum(hm, 1, keepdims=True) / N
    v = jnp.sum(hm * hm, 1, keepdims=True) / N - m * m
    x2_o[...] = jax.nn.sigmoid(
        g2[...] * (h - m) / jnp.sqrt(v + EPS) + b2[...])


_tc2 = pl.pallas_call(
    _tc2_body,
    out_shape=jax.ShapeDtypeStruct((1, NPAD), jnp.float32),
)


def _tc3_body(p3p, cntc, x1T, x2, we3c, root3c, bias3c, g3c, b3c, out_o):
    R3c = jnp.maximum(we3c[...], 0.0)
    RTc = root3c[...]
    z = jnp.zeros((F, 1), jnp.float32)

    def hchunk(l0):
        lsl = pl.ds(l0, LB)
        p3n = (p3p[0, :, lsl] + p3p[1, :, lsl]) / cntc[:, lsl]
        return lsl, R3c * p3n + RTc * x2[:, lsl] + bias3c[...]

    def pass1(i, carry):
        sh, sq = carry
        l0 = i * LB
        _, hT = hchunk(l0)
        hm = hT * _lane_mask(l0)
        return (sh + jnp.sum(hm, 1, keepdims=True),
                sq + jnp.sum(hm * hm, 1, keepdims=True))

    sh, sq = lax.fori_loop(0, NLCH, pass1, (z, z))
    m = sh / N
    v = sq / N - m * m
    scale = g3c[...] / jnp.sqrt(v + EPS)
    off = b3c[...] - m * scale

    def pass2(i, carry):
        lsl, hT = hchunk(i * LB)
        x3aT = jax.nn.sigmoid(hT * scale + off)
        out_o[:, lsl] = (x3aT + x1T[:, lsl]) * 0.5
        return carry

    lax.fori_loop(0, NLCH, pass2, 0)


_tc3 = pl.pallas_call(
    _tc3_body,
    out_shape=jax.ShapeDtypeStruct((F, NPAD), jnp.float32),
)


def _pad2(w, rows, cols):
    return jnp.pad(w, ((0, rows - w.shape[0]), (0, cols - w.shape[1])))


def kernel(x, pos_edge_index, edge_attr,
           We1, be1, root1, bias1, g1, b1,
           We2, be2, root2, bias2, g2, b2,
           We3, be3, root3, bias3, g3, b3):
    f32 = jnp.float32
    del be1, be2, be3

    pad_e = EPAD - E
    fake = (jnp.arange(pad_e, dtype=jnp.int32) % (NPAD - N)) + N
    pei = jnp.concatenate(
        [pos_edge_index.astype(jnp.int32),
         jnp.broadcast_to(fake, (2, pad_e))],
        axis=1).reshape(2, EPAD // SUB, SUB)
    ap = jnp.pad(edge_attr.reshape(E).astype(f32), (0, pad_e))
    onesp = jnp.pad(jnp.ones((E,), f32), (0, pad_e))

    xpad = jnp.pad(x, ((0, NPAD - N), (0, F - 35)))
    z48 = jnp.zeros((NPAD, F), f32)
    z1 = jnp.zeros((NPAD,), f32)

    w1t = _pad2(We1.reshape(35, 35).T, F, F)
    root1t = _pad2(root1.T, F, F)
    bias1c = jnp.pad(bias1, (0, F - 35)).reshape(F, 1)
    g1c = jnp.pad(g1, (0, F - 35)).reshape(F, 1)
    b1c = jnp.pad(b1, (0, F - 35)).reshape(F, 1)
    we2r = _pad2(We2, 1, F)
    root2t = _pad2(root2.reshape(1, 35), 1, F)
    bias2r = bias2.reshape(1, 1)
    g2r = g2.reshape(1, 1)
    b2r = b2.reshape(1, 1)
    we3c = _pad2(We3.reshape(35, 1), F, 1)
    root3c = _pad2(root3.reshape(35, 1), F, 1)
    bias3c = jnp.pad(bias3, (0, F - 35)).reshape(F, 1)
    g3c = jnp.pad(g3, (0, F - 35)).reshape(F, 1)
    b3c = jnp.pad(b3, (0, F - 35)).reshape(F, 1)

    P1p, cntp = _sc_rows(pei, ap, xpad, z48, z1, onesp)
    x1T, y1, cntc = _tc1(P1p, cntp, xpad,
                         w1t, root1t, bias1c, g1c, b1c, we2r)
    p2p = _sc_scalar(pei, ap, y1, z1)
    x2 = _tc2(p2p, cntc, x1T, root2t, bias2r, g2r, b2r)
    p3p = _sc_scalar(pei, ap, x2, z1)
    outT = _tc3(p3p, cntc, x1T, x2, we3c, root3c, bias3c, g3c, b3c)
    return outT[:35, :N].T
